# baseline (speedup 1.0000x reference)
import jax
import jax.numpy as jnp
import numpy as np
from jax import lax
from jax.experimental import pallas as pl
from jax.experimental.pallas import tpu as pltpu

_ROWS = 8
_W = 1024
_THREEFRY_C = 0x1BD11BDA
_F32_ONE_BITS = 0x3F800000
_TINY = np.float32(np.finfo(np.float32).tiny)
_NEG_INF = np.float32(-np.inf)
_INT_MAX = np.int32(2**31 - 1)


def _rotl(x, r):
    return (x << jnp.int32(r)) | lax.shift_right_logical(x, jnp.int32(32 - r))


def _threefry2x32(k0, k1, c1):
    k2 = k0 ^ k1 ^ jnp.int32(_THREEFRY_C)
    x0 = k0
    x1 = c1 + k1
    ks = (k0, k1, k2)
    rots = ((13, 15, 26, 6), (17, 29, 16, 24),
            (13, 15, 26, 6), (17, 29, 16, 24), (13, 15, 26, 6))
    for d in range(5):
        for r in rots[d]:
            x0 = x0 + x1
            x1 = _rotl(x1, r) ^ x0
        x0 = x0 + ks[(d + 1) % 3]
        x1 = x1 + ks[(d + 2) % 3] + jnp.int32(d + 1)
    return x0 ^ x1


def _bits_to_unit_float(bits):
    fb = lax.shift_right_logical(bits, jnp.int32(9)) | jnp.int32(_F32_ONE_BITS)
    return lax.bitcast_convert_type(fb, jnp.float32) - jnp.float32(1.0)


def _make_scan_body(V):
    def _body(scal_ref, logits_ref, acc_s_ref, acc_c_ref):
        i = pl.program_id(0)
        j = pl.program_id(1)

        x = logits_ref[...]
        R, W = x.shape

        rowoff = (lax.broadcasted_iota(jnp.int32, (R, 1), 0) + i * R) * V
        col = lax.broadcasted_iota(jnp.int32, (R, W), 1) + j * W
        bits = _threefry2x32(scal_ref[0], scal_ref[1], rowoff + col)
        floats = _bits_to_unit_float(bits)
        u = jnp.maximum(_TINY, floats * (jnp.float32(1.0) - _TINY) + _TINY)
        g = -jnp.log(-jnp.log(u))

        score = jnp.where(col < V, x + g, _NEG_INF)

        take = jnp.logical_or(score > acc_s_ref[...], j == 0)
        acc_s_ref[...] = jnp.where(take, score, acc_s_ref[...])
        acc_c_ref[...] = jnp.where(take, col, acc_c_ref[...])

    return _body


def _finish_body(scal_ref, acc_s_ref, acc_c_ref, target_ref, out_ref):
    i = pl.program_id(0)
    a_s = acc_s_ref[...]
    a_c = acc_c_ref[...]
    R = a_s.shape[0]

    best = jnp.max(a_s, axis=1, keepdims=True)
    idx = jnp.min(jnp.where(a_s == best, a_c, _INT_MAX), axis=1, keepdims=True)
    sample = idx.astype(jnp.float32)

    rctr = lax.broadcasted_iota(jnp.int32, (R, 1), 0) + i * R
    cbits = _threefry2x32(scal_ref[2], scal_ref[3], rctr)
    cp = jnp.maximum(jnp.float32(0.0), _bits_to_unit_float(cbits))

    t = target_ref[...]
    tcol = lax.broadcasted_iota(jnp.int32, t.shape, 1)
    tgt = jnp.sum(jnp.where(tcol == scal_ref[4], t, jnp.float32(0.0)),
                  axis=1, keepdims=True)

    thr = lax.bitcast_convert_type(scal_ref[5], jnp.float32)
    out_ref[...] = jnp.where(cp < thr, tgt, sample)


def kernel(target, logits, step, summary_step):
    B, V = logits.shape
    T = target.shape[1]

    skd = lax.bitcast_convert_type(
        jax.random.key_data(jax.random.fold_in(jax.random.key(42), summary_step)),
        jnp.int32)
    ckd = lax.bitcast_convert_type(
        jax.random.key_data(jax.random.fold_in(jax.random.key(7), step)),
        jnp.int32)
    stepf = jnp.asarray(step, jnp.float32)
    thr = jnp.float32(100.0) / (jnp.float32(100.0) + jnp.exp(stepf / jnp.float32(100.0)))
    scalars = jnp.concatenate([
        skd.reshape(2), ckd.reshape(2),
        jnp.asarray(step, jnp.int32).reshape(1),
        lax.bitcast_convert_type(thr, jnp.int32).reshape(1),
    ])

    acc_s, acc_c = pl.pallas_call(
        _make_scan_body(V),
        grid=(B // _ROWS, pl.cdiv(V, _W)),
        in_specs=[
            pl.BlockSpec(memory_space=pltpu.SMEM),
            pl.BlockSpec((_ROWS, _W), lambda i, j: (i, j)),
        ],
        out_specs=[
            pl.BlockSpec((_ROWS, _W), lambda i, j: (i, 0)),
            pl.BlockSpec((_ROWS, _W), lambda i, j: (i, 0)),
        ],
        out_shape=[
            jax.ShapeDtypeStruct((B, _W), jnp.float32),
            jax.ShapeDtypeStruct((B, _W), jnp.int32),
        ],
    )(scalars, logits)

    out = pl.pallas_call(
        _finish_body,
        grid=(B // _ROWS,),
        in_specs=[
            pl.BlockSpec(memory_space=pltpu.SMEM),
            pl.BlockSpec((_ROWS, _W), lambda i: (i, 0)),
            pl.BlockSpec((_ROWS, _W), lambda i: (i, 0)),
            pl.BlockSpec((_ROWS, T), lambda i: (i, 0)),
        ],
        out_specs=pl.BlockSpec((_ROWS, 1), lambda i: (i, 0)),
        out_shape=jax.ShapeDtypeStruct((B, 1), jnp.float32),
    )(scalars, acc_s, acc_c, target)
    return out


# speedup vs baseline: 2.8338x; 2.8338x over previous
import jax
import jax.numpy as jnp
import numpy as np
from jax import lax
from jax.experimental import pallas as pl
from jax.experimental.pallas import tpu as pltpu

_ROWS = 8
_W = 1024
_THREEFRY_C = 0x1BD11BDA
_F32_ONE_BITS = 0x3F800000
_TINY = np.float32(np.finfo(np.float32).tiny)
_NEG_INF = np.float32(-np.inf)
_INT_MAX = np.int32(2**31 - 1)


def _rotl(x, r):
    return (x << jnp.int32(r)) | lax.shift_right_logical(x, jnp.int32(32 - r))


def _threefry2x32(k0, k1, c1):
    k2 = k0 ^ k1 ^ jnp.int32(_THREEFRY_C)
    x0 = k0
    x1 = c1 + k1
    ks = (k0, k1, k2)
    rots = ((13, 15, 26, 6), (17, 29, 16, 24),
            (13, 15, 26, 6), (17, 29, 16, 24), (13, 15, 26, 6))
    for d in range(5):
        for r in rots[d]:
            x0 = x0 + x1
            x1 = _rotl(x1, r) ^ x0
        x0 = x0 + ks[(d + 1) % 3]
        x1 = x1 + ks[(d + 2) % 3] + jnp.int32(d + 1)
    return x0 ^ x1


def _bits_to_unit_float(bits):
    fb = lax.shift_right_logical(bits, jnp.int32(9)) | jnp.int32(_F32_ONE_BITS)
    return lax.bitcast_convert_type(fb, jnp.float32) - jnp.float32(1.0)


def _make_body(V, nch):
    def _body(scal_ref, logits_ref, target_ref, out_ref):
        i = pl.program_id(0)
        R = _ROWS
        W = _W
        k0 = scal_ref[0]
        k1 = scal_ref[1]
        rowoff = (lax.broadcasted_iota(jnp.int32, (R, 1), 0) + i * R) * V
        lane = lax.broadcasted_iota(jnp.int32, (R, W), 1)

        def chunk(j, carry):
            acc_s, acc_c = carry
            x = logits_ref[:, pl.ds(j * W, W)]
            col = lane + j * W
            bits = _threefry2x32(k0, k1, rowoff + col)
            floats = _bits_to_unit_float(bits)
            u = jnp.maximum(_TINY, floats * (jnp.float32(1.0) - _TINY) + _TINY)
            g = -jnp.log(-jnp.log(u))
            score = jnp.where(col < V, x + g, _NEG_INF)
            take = score > acc_s
            return (jnp.where(take, score, acc_s),
                    jnp.where(take, col, acc_c))

        acc0 = (jnp.full((R, W), _NEG_INF, jnp.float32),
                jnp.full((R, W), _INT_MAX, jnp.int32))
        a_s, a_c = lax.fori_loop(0, nch, chunk, acc0)

        best = jnp.max(a_s, axis=1, keepdims=True)
        idx = jnp.min(jnp.where(a_s == best, a_c, _INT_MAX),
                      axis=1, keepdims=True)
        sample = idx.astype(jnp.float32)

        rctr = lax.broadcasted_iota(jnp.int32, (R, 1), 0) + i * R
        cbits = _threefry2x32(scal_ref[2], scal_ref[3], rctr)
        cp = jnp.maximum(jnp.float32(0.0), _bits_to_unit_float(cbits))

        t = target_ref[...]
        tcol = lax.broadcasted_iota(jnp.int32, t.shape, 1)
        tgt = jnp.sum(jnp.where(tcol == scal_ref[4], t, jnp.float32(0.0)),
                      axis=1, keepdims=True)

        thr = lax.bitcast_convert_type(scal_ref[5], jnp.float32)
        out_ref[...] = jnp.where(cp < thr, tgt, sample)

    return _body


def kernel(target, logits, step, summary_step):
    B, V = logits.shape
    T = target.shape[1]

    skd = lax.bitcast_convert_type(
        jax.random.key_data(jax.random.fold_in(jax.random.key(42), summary_step)),
        jnp.int32)
    ckd = lax.bitcast_convert_type(
        jax.random.key_data(jax.random.fold_in(jax.random.key(7), step)),
        jnp.int32)
    stepf = jnp.asarray(step, jnp.float32)
    thr = jnp.float32(100.0) / (jnp.float32(100.0) + jnp.exp(stepf / jnp.float32(100.0)))
    scalars = jnp.concatenate([
        skd.reshape(2), ckd.reshape(2),
        jnp.asarray(step, jnp.int32).reshape(1),
        lax.bitcast_convert_type(thr, jnp.int32).reshape(1),
    ])

    nch = pl.cdiv(V, _W)
    vpad = nch * _W
    out = pl.pallas_call(
        _make_body(V, nch),
        grid=(B // _ROWS,),
        in_specs=[
            pl.BlockSpec(memory_space=pltpu.SMEM),
            pl.BlockSpec((_ROWS, vpad), lambda i: (i, 0)),
            pl.BlockSpec((_ROWS, T), lambda i: (i, 0)),
        ],
        out_specs=pl.BlockSpec((_ROWS, 1), lambda i: (i, 0)),
        out_shape=jax.ShapeDtypeStruct((B, 1), jnp.float32),
    )(scalars, logits, target)
    return out


# speedup vs baseline: 3.0049x; 1.0604x over previous
import jax
import jax.numpy as jnp
import numpy as np
from jax import lax
from jax.experimental import pallas as pl
from jax.experimental.pallas import tpu as pltpu

_ROWS = 8
_W = 1024
_THREEFRY_C = 0x1BD11BDA
_F32_ONE_BITS = 0x3F800000
_TINY = np.float32(np.finfo(np.float32).tiny)
_NEG_INF = np.float32(-np.inf)
_INT_MAX = np.int32(2**31 - 1)


def _rotl(x, r):
    return (x << jnp.int32(r)) | lax.shift_right_logical(x, jnp.int32(32 - r))


def _threefry2x32(k0, k1, c1):
    k2 = k0 ^ k1 ^ jnp.int32(_THREEFRY_C)
    x0 = k0
    x1 = c1 + k1
    ks = (k0, k1, k2)
    rots = ((13, 15, 26, 6), (17, 29, 16, 24),
            (13, 15, 26, 6), (17, 29, 16, 24), (13, 15, 26, 6))
    for d in range(5):
        for r in rots[d]:
            x0 = x0 + x1
            x1 = _rotl(x1, r) ^ x0
        x0 = x0 + ks[(d + 1) % 3]
        x1 = x1 + ks[(d + 2) % 3] + jnp.int32(d + 1)
    return x0 ^ x1


def _bits_to_unit_float(bits):
    fb = lax.shift_right_logical(bits, jnp.int32(9)) | jnp.int32(_F32_ONE_BITS)
    return lax.bitcast_convert_type(fb, jnp.float32) - jnp.float32(1.0)


def _make_body(V, nch):
    def _body(scal_ref, logits_ref, target_ref, out_ref):
        i = pl.program_id(0)
        R = _ROWS
        W = _W
        k0 = scal_ref[0]
        k1 = scal_ref[1]
        rowoff = (lax.broadcasted_iota(jnp.int32, (R, 1), 0) + i * R) * V
        lane = lax.broadcasted_iota(jnp.int32, (R, W), 1)

        def score_of(j, masked):
            x = logits_ref[:, pl.ds(j * W, W)]
            col = lane + j * W
            bits = _threefry2x32(k0, k1, rowoff + col)
            floats = _bits_to_unit_float(bits)
            u = jnp.maximum(_TINY, floats * (jnp.float32(1.0) - _TINY) + _TINY)
            g = -jnp.log(-jnp.log(u))
            score = x + g
            if masked:
                score = jnp.where(col < V, score, _NEG_INF)
            return score, col

        def merge(carry, score, col):
            acc_s, acc_c = carry
            take = score > acc_s
            return (jnp.where(take, score, acc_s),
                    jnp.where(take, col, acc_c))

        def chunk2(jj, carry):
            s0, c0 = score_of(jj * 2, False)
            carry = merge(carry, s0, c0)
            s1, c1 = score_of(jj * 2 + 1, False)
            return merge(carry, s1, c1)

        acc0 = (jnp.full((R, W), _NEG_INF, jnp.float32),
                jnp.full((R, W), _INT_MAX, jnp.int32))
        carry = lax.fori_loop(0, (nch - 1) // 2, chunk2, acc0)
        for j in range(2 * ((nch - 1) // 2), nch):
            s, c = score_of(j, masked=(j == nch - 1))
            carry = merge(carry, s, c)
        a_s, a_c = carry

        best = jnp.max(a_s, axis=1, keepdims=True)
        idx = jnp.min(jnp.where(a_s == best, a_c, _INT_MAX),
                      axis=1, keepdims=True)
        sample = idx.astype(jnp.float32)

        rctr = lax.broadcasted_iota(jnp.int32, (R, 1), 0) + i * R
        cbits = _threefry2x32(scal_ref[2], scal_ref[3], rctr)
        cp = jnp.maximum(jnp.float32(0.0), _bits_to_unit_float(cbits))

        t = target_ref[...]
        tcol = lax.broadcasted_iota(jnp.int32, t.shape, 1)
        tgt = jnp.sum(jnp.where(tcol == scal_ref[4], t, jnp.float32(0.0)),
                      axis=1, keepdims=True)

        thr = lax.bitcast_convert_type(scal_ref[5], jnp.float32)
        out_ref[...] = jnp.where(cp < thr, tgt, sample)

    return _body


def kernel(target, logits, step, summary_step):
    B, V = logits.shape
    T = target.shape[1]

    skd = lax.bitcast_convert_type(
        jax.random.key_data(jax.random.fold_in(jax.random.key(42), summary_step)),
        jnp.int32)
    ckd = lax.bitcast_convert_type(
        jax.random.key_data(jax.random.fold_in(jax.random.key(7), step)),
        jnp.int32)
    stepf = jnp.asarray(step, jnp.float32)
    thr = jnp.float32(100.0) / (jnp.float32(100.0) + jnp.exp(stepf / jnp.float32(100.0)))
    scalars = jnp.concatenate([
        skd.reshape(2), ckd.reshape(2),
        jnp.asarray(step, jnp.int32).reshape(1),
        lax.bitcast_convert_type(thr, jnp.int32).reshape(1),
    ])

    nch = pl.cdiv(V, _W)
    vpad = nch * _W
    out = pl.pallas_call(
        _make_body(V, nch),
        grid=(B // _ROWS,),
        in_specs=[
            pl.BlockSpec(memory_space=pltpu.SMEM),
            pl.BlockSpec((_ROWS, vpad), lambda i: (i, 0)),
            pl.BlockSpec((_ROWS, T), lambda i: (i, 0)),
        ],
        out_specs=pl.BlockSpec((_ROWS, 1), lambda i: (i, 0)),
        out_shape=jax.ShapeDtypeStruct((B, 1), jnp.float32),
    )(scalars, logits, target)
    return out


# speedup vs baseline: 3.2912x; 1.0953x over previous
import functools

import jax
import jax.numpy as jnp
import numpy as np
from jax import lax
from jax.experimental import pallas as pl
from jax.experimental.pallas import tpu as pltpu
from jax.experimental.pallas import tpu_sc as plsc

_ROWS = 8
_W = 1024
_SC_ROWS = 32
_SC_CW = 2000
_THREEFRY_C = 0x1BD11BDA
_F32_ONE_BITS = 0x3F800000
_TINY = np.float32(np.finfo(np.float32).tiny)
_NEG_INF = np.float32(-np.inf)
_INT_MAX = np.int32(2**31 - 1)


def _rotl(x, r):
    return (x << jnp.int32(r)) | lax.shift_right_logical(x, jnp.int32(32 - r))


def _threefry2x32(k0, k1, c1):
    k2 = k0 ^ k1 ^ jnp.int32(_THREEFRY_C)
    x0 = k0
    x1 = c1 + k1
    ks = (k0, k1, k2)
    rots = ((13, 15, 26, 6), (17, 29, 16, 24),
            (13, 15, 26, 6), (17, 29, 16, 24), (13, 15, 26, 6))
    for d in range(5):
        for r in rots[d]:
            x0 = x0 + x1
            x1 = _rotl(x1, r) ^ x0
        x0 = x0 + ks[(d + 1) % 3]
        x1 = x1 + ks[(d + 2) % 3] + jnp.int32(d + 1)
    return x0 ^ x1


def _bits_to_unit_float(bits):
    fb = lax.shift_right_logical(bits, jnp.int32(9)) | jnp.int32(_F32_ONE_BITS)
    return lax.bitcast_convert_type(fb, jnp.float32) - jnp.float32(1.0)


def _sc_bits(keys, V, vpad, row0):
    mesh = plsc.VectorSubcoreMesh(core_axis_name="c", subcore_axis_name="s")
    nstripes = vpad // 128
    per_w = (nstripes + 31) // 32

    @functools.partial(
        pl.kernel,
        out_type=jax.ShapeDtypeStruct((_SC_ROWS, vpad), jnp.int32),
        mesh=mesh,
        scratch_types=[
            pltpu.VMEM((_SC_ROWS, 128), jnp.int32),
            pltpu.VMEM((2, 16), jnp.int32),
        ],
    )
    def k(keys_hbm, out_hbm, buf, kbuf):
        wid = lax.axis_index("s") * 2 + lax.axis_index("c")
        pltpu.sync_copy(keys_hbm, kbuf)
        k0 = kbuf.at[0][...]
        k1 = kbuf.at[1][...]
        lane = lax.iota(jnp.int32, 16)

        @pl.loop(0, per_w)
        def _stripe(it):
            s = wid + it * 32

            @pl.when(s < nstripes)
            def _do():
                c0 = pl.multiple_of(s * 128, 128)

                @pl.loop(0, _SC_ROWS)
                def _row(r):
                    base = (row0 + r) * V + c0

                    for u in range(8):
                        ctr = (base + u * 16) + lane
                        buf.at[r, pl.ds(u * 16, 16)][...] = _threefry2x32(
                            k0, k1, ctr)

                pltpu.sync_copy(buf, out_hbm.at[:, pl.ds(c0, 128)])

    return k(keys)


def _make_tc_body(V, nch, blk0, with_bits):
    def _body(scal_ref, logits_ref, target_ref, *rest):
        if with_bits:
            bits_ref, out_ref = rest
        else:
            (out_ref,) = rest
        i = pl.program_id(0) + blk0
        R = _ROWS
        W = _W
        k0 = scal_ref[0]
        k1 = scal_ref[1]
        rowoff = (lax.broadcasted_iota(jnp.int32, (R, 1), 0) + i * R) * V
        lane = lax.broadcasted_iota(jnp.int32, (R, W), 1)

        def score_of(j, masked):
            x = logits_ref[:, pl.ds(j * W, W)]
            col = lane + j * W
            if with_bits:
                bits = bits_ref[:, pl.ds(j * W, W)]
            else:
                bits = _threefry2x32(k0, k1, rowoff + col)
            floats = _bits_to_unit_float(bits)
            u = jnp.maximum(_TINY, floats * (jnp.float32(1.0) - _TINY) + _TINY)
            g = -jnp.log(-jnp.log(u))
            score = x + g
            if masked:
                score = jnp.where(col < V, score, _NEG_INF)
            return score, col

        def merge(carry, score, col):
            acc_s, acc_c = carry
            take = score > acc_s
            return (jnp.where(take, score, acc_s),
                    jnp.where(take, col, acc_c))

        def chunk2(jj, carry):
            s0, c0 = score_of(jj * 2, False)
            carry = merge(carry, s0, c0)
            s1, c1 = score_of(jj * 2 + 1, False)
            return merge(carry, s1, c1)

        acc0 = (jnp.full((R, W), _NEG_INF, jnp.float32),
                jnp.full((R, W), _INT_MAX, jnp.int32))
        carry = lax.fori_loop(0, (nch - 1) // 2, chunk2, acc0)
        for j in range(2 * ((nch - 1) // 2), nch):
            s, c = score_of(j, masked=(j == nch - 1))
            carry = merge(carry, s, c)
        a_s, a_c = carry

        best = jnp.max(a_s, axis=1, keepdims=True)
        idx = jnp.min(jnp.where(a_s == best, a_c, _INT_MAX),
                      axis=1, keepdims=True)
        sample = idx.astype(jnp.float32)

        rctr = lax.broadcasted_iota(jnp.int32, (R, 1), 0) + i * R
        cbits = _threefry2x32(scal_ref[2], scal_ref[3], rctr)
        cp = jnp.maximum(jnp.float32(0.0), _bits_to_unit_float(cbits))

        t = target_ref[...]
        tcol = lax.broadcasted_iota(jnp.int32, t.shape, 1)
        tgt = jnp.sum(jnp.where(tcol == scal_ref[4], t, jnp.float32(0.0)),
                      axis=1, keepdims=True)

        thr = lax.bitcast_convert_type(scal_ref[5], jnp.float32)
        out_ref[...] = jnp.where(cp < thr, tgt, sample)

    return _body


def kernel(target, logits, step, summary_step):
    B, V = logits.shape
    T = target.shape[1]
    tc_rows = B - _SC_ROWS
    tc_blocks = tc_rows // _ROWS
    sc_blocks = _SC_ROWS // _ROWS

    skd = lax.bitcast_convert_type(
        jax.random.key_data(jax.random.fold_in(jax.random.key(42), summary_step)),
        jnp.int32)
    ckd = lax.bitcast_convert_type(
        jax.random.key_data(jax.random.fold_in(jax.random.key(7), step)),
        jnp.int32)
    stepf = jnp.asarray(step, jnp.float32)
    thr = jnp.float32(100.0) / (jnp.float32(100.0) + jnp.exp(stepf / jnp.float32(100.0)))
    scalars = jnp.concatenate([
        skd.reshape(2), ckd.reshape(2),
        jnp.asarray(step, jnp.int32).reshape(1),
        lax.bitcast_convert_type(thr, jnp.int32).reshape(1),
    ])

    nch = pl.cdiv(V, _W)
    vpad = nch * _W

    keys16 = jnp.tile(skd.reshape(2, 1), (1, 16))
    sc_bits = _sc_bits(keys16, V, vpad, tc_rows)

    out_a = pl.pallas_call(
        _make_tc_body(V, nch, 0, False),
        grid=(tc_blocks,),
        in_specs=[
            pl.BlockSpec(memory_space=pltpu.SMEM),
            pl.BlockSpec((_ROWS, vpad), lambda i: (i, 0)),
            pl.BlockSpec((_ROWS, T), lambda i: (i, 0)),
        ],
        out_specs=pl.BlockSpec((_ROWS, 1), lambda i: (i, 0)),
        out_shape=jax.ShapeDtypeStruct((tc_rows, 1), jnp.float32),
    )(scalars, logits, target)

    out_b = pl.pallas_call(
        _make_tc_body(V, nch, tc_blocks, True),
        grid=(sc_blocks,),
        in_specs=[
            pl.BlockSpec(memory_space=pltpu.SMEM),
            pl.BlockSpec((_ROWS, vpad), lambda i, b=tc_blocks: (i + b, 0)),
            pl.BlockSpec((_ROWS, T), lambda i, b=tc_blocks: (i + b, 0)),
            pl.BlockSpec((_ROWS, vpad), lambda i: (i, 0)),
        ],
        out_specs=pl.BlockSpec((_ROWS, 1), lambda i: (i, 0)),
        out_shape=jax.ShapeDtypeStruct((_SC_ROWS, 1), jnp.float32),
    )(scalars, logits, target, sc_bits)

    return jnp.concatenate([out_a, out_b], axis=0)


# speedup vs baseline: 3.3099x; 1.0057x over previous
import functools

import jax
import jax.numpy as jnp
import numpy as np
from jax import lax
from jax.experimental import pallas as pl
from jax.experimental.pallas import tpu as pltpu
from jax.experimental.pallas import tpu_sc as plsc

_ROWS = 8
_W = 1024
_SC_ROWS = 48
_SC_CW = 2000
_THREEFRY_C = 0x1BD11BDA
_F32_ONE_BITS = 0x3F800000
_TINY = np.float32(np.finfo(np.float32).tiny)
_NEG_INF = np.float32(-np.inf)
_INT_MAX = np.int32(2**31 - 1)


def _rotl(x, r):
    return (x << jnp.int32(r)) | lax.shift_right_logical(x, jnp.int32(32 - r))


def _threefry2x32(k0, k1, c1):
    k2 = k0 ^ k1 ^ jnp.int32(_THREEFRY_C)
    x0 = k0
    x1 = c1 + k1
    ks = (k0, k1, k2)
    rots = ((13, 15, 26, 6), (17, 29, 16, 24),
            (13, 15, 26, 6), (17, 29, 16, 24), (13, 15, 26, 6))
    for d in range(5):
        for r in rots[d]:
            x0 = x0 + x1
            x1 = _rotl(x1, r) ^ x0
        x0 = x0 + ks[(d + 1) % 3]
        x1 = x1 + ks[(d + 2) % 3] + jnp.int32(d + 1)
    return x0 ^ x1


def _bits_to_unit_float(bits):
    fb = lax.shift_right_logical(bits, jnp.int32(9)) | jnp.int32(_F32_ONE_BITS)
    return lax.bitcast_convert_type(fb, jnp.float32) - jnp.float32(1.0)


def _sc_bits(keys, V, vpad, row0):
    mesh = plsc.VectorSubcoreMesh(core_axis_name="c", subcore_axis_name="s")
    nstripes = vpad // 128
    per_w = (nstripes + 31) // 32

    @functools.partial(
        pl.kernel,
        out_type=jax.ShapeDtypeStruct((_SC_ROWS, vpad), jnp.int32),
        mesh=mesh,
        scratch_types=[
            pltpu.VMEM((_SC_ROWS, 128), jnp.int32),
            pltpu.VMEM((2, 16), jnp.int32),
        ],
    )
    def k(keys_hbm, out_hbm, buf, kbuf):
        wid = lax.axis_index("s") * 2 + lax.axis_index("c")
        pltpu.sync_copy(keys_hbm, kbuf)
        k0 = kbuf.at[0][...]
        k1 = kbuf.at[1][...]
        lane = lax.iota(jnp.int32, 16)

        @pl.loop(0, per_w)
        def _stripe(it):
            s = wid + it * 32

            @pl.when(s < nstripes)
            def _do():
                c0 = pl.multiple_of(s * 128, 128)

                @pl.loop(0, _SC_ROWS)
                def _row(r):
                    base = (row0 + r) * V + c0

                    for u in range(8):
                        ctr = (base + u * 16) + lane
                        buf.at[r, pl.ds(u * 16, 16)][...] = _threefry2x32(
                            k0, k1, ctr)

                pltpu.sync_copy(buf, out_hbm.at[:, pl.ds(c0, 128)])

    return k(keys)


def _make_tc_body(V, nch, blk0, with_bits):
    def _body(scal_ref, logits_ref, target_ref, *rest):
        if with_bits:
            bits_ref, out_ref = rest
        else:
            (out_ref,) = rest
        i = pl.program_id(0) + blk0
        R = _ROWS
        W = _W
        k0 = scal_ref[0]
        k1 = scal_ref[1]
        rowoff = (lax.broadcasted_iota(jnp.int32, (R, 1), 0) + i * R) * V
        lane = lax.broadcasted_iota(jnp.int32, (R, W), 1)

        def score_of(j, masked):
            x = logits_ref[:, pl.ds(j * W, W)]
            col = lane + j * W
            if with_bits:
                bits = bits_ref[:, pl.ds(j * W, W)]
            else:
                bits = _threefry2x32(k0, k1, rowoff + col)
            floats = _bits_to_unit_float(bits)
            u = jnp.maximum(_TINY, floats * (jnp.float32(1.0) - _TINY) + _TINY)
            g = -jnp.log(-jnp.log(u))
            score = x + g
            if masked:
                score = jnp.where(col < V, score, _NEG_INF)
            return score, col

        def merge(carry, score, col):
            acc_s, acc_c = carry
            take = score > acc_s
            return (jnp.where(take, score, acc_s),
                    jnp.where(take, col, acc_c))

        def chunk2(jj, carry):
            s0, c0 = score_of(jj * 2, False)
            carry = merge(carry, s0, c0)
            s1, c1 = score_of(jj * 2 + 1, False)
            return merge(carry, s1, c1)

        acc0 = (jnp.full((R, W), _NEG_INF, jnp.float32),
                jnp.full((R, W), _INT_MAX, jnp.int32))
        carry = lax.fori_loop(0, (nch - 1) // 2, chunk2, acc0)
        for j in range(2 * ((nch - 1) // 2), nch):
            s, c = score_of(j, masked=(j == nch - 1))
            carry = merge(carry, s, c)
        a_s, a_c = carry

        best = jnp.max(a_s, axis=1, keepdims=True)
        idx = jnp.min(jnp.where(a_s == best, a_c, _INT_MAX),
                      axis=1, keepdims=True)
        sample = idx.astype(jnp.float32)

        rctr = lax.broadcasted_iota(jnp.int32, (R, 1), 0) + i * R
        cbits = _threefry2x32(scal_ref[2], scal_ref[3], rctr)
        cp = jnp.maximum(jnp.float32(0.0), _bits_to_unit_float(cbits))

        t = target_ref[...]
        tcol = lax.broadcasted_iota(jnp.int32, t.shape, 1)
        tgt = jnp.sum(jnp.where(tcol == scal_ref[4], t, jnp.float32(0.0)),
                      axis=1, keepdims=True)

        thr = lax.bitcast_convert_type(scal_ref[5], jnp.float32)
        out_ref[...] = jnp.where(cp < thr, tgt, sample)

    return _body


def kernel(target, logits, step, summary_step):
    B, V = logits.shape
    T = target.shape[1]
    tc_rows = B - _SC_ROWS
    tc_blocks = tc_rows // _ROWS
    sc_blocks = _SC_ROWS // _ROWS

    skd = lax.bitcast_convert_type(
        jax.random.key_data(jax.random.fold_in(jax.random.key(42), summary_step)),
        jnp.int32)
    ckd = lax.bitcast_convert_type(
        jax.random.key_data(jax.random.fold_in(jax.random.key(7), step)),
        jnp.int32)
    stepf = jnp.asarray(step, jnp.float32)
    thr = jnp.float32(100.0) / (jnp.float32(100.0) + jnp.exp(stepf / jnp.float32(100.0)))
    scalars = jnp.concatenate([
        skd.reshape(2), ckd.reshape(2),
        jnp.asarray(step, jnp.int32).reshape(1),
        lax.bitcast_convert_type(thr, jnp.int32).reshape(1),
    ])

    nch = pl.cdiv(V, _W)
    vpad = nch * _W

    keys16 = jnp.tile(skd.reshape(2, 1), (1, 16))
    sc_bits = _sc_bits(keys16, V, vpad, tc_rows)

    out_a = pl.pallas_call(
        _make_tc_body(V, nch, 0, False),
        grid=(tc_blocks,),
        in_specs=[
            pl.BlockSpec(memory_space=pltpu.SMEM),
            pl.BlockSpec((_ROWS, vpad), lambda i: (i, 0)),
            pl.BlockSpec((_ROWS, T), lambda i: (i, 0)),
        ],
        out_specs=pl.BlockSpec((_ROWS, 1), lambda i: (i, 0)),
        out_shape=jax.ShapeDtypeStruct((tc_rows, 1), jnp.float32),
    )(scalars, logits, target)

    out_b = pl.pallas_call(
        _make_tc_body(V, nch, tc_blocks, True),
        grid=(sc_blocks,),
        in_specs=[
            pl.BlockSpec(memory_space=pltpu.SMEM),
            pl.BlockSpec((_ROWS, vpad), lambda i, b=tc_blocks: (i + b, 0)),
            pl.BlockSpec((_ROWS, T), lambda i, b=tc_blocks: (i + b, 0)),
            pl.BlockSpec((_ROWS, vpad), lambda i: (i, 0)),
        ],
        out_specs=pl.BlockSpec((_ROWS, 1), lambda i: (i, 0)),
        out_shape=jax.ShapeDtypeStruct((_SC_ROWS, 1), jnp.float32),
    )(scalars, logits, target, sc_bits)

    return jnp.concatenate([out_a, out_b], axis=0)


# speedup vs baseline: 3.4473x; 1.0415x over previous
import functools

import jax
import jax.numpy as jnp
import numpy as np
from jax import lax
from jax.experimental import pallas as pl
from jax.experimental.pallas import tpu as pltpu
from jax.experimental.pallas import tpu_sc as plsc

_ROWS = 8
_W = 1024
_SC_ROWS = 40
_SC_CW = 2000
_THREEFRY_C = 0x1BD11BDA
_F32_ONE_BITS = 0x3F800000
_TINY = np.float32(np.finfo(np.float32).tiny)
_NEG_INF = np.float32(-np.inf)
_INT_MAX = np.int32(2**31 - 1)


def _rotl(x, r):
    return (x << jnp.int32(r)) | lax.shift_right_logical(x, jnp.int32(32 - r))


def _threefry2x32(k0, k1, c1):
    k2 = k0 ^ k1 ^ jnp.int32(_THREEFRY_C)
    x0 = k0
    x1 = c1 + k1
    ks = (k0, k1, k2)
    rots = ((13, 15, 26, 6), (17, 29, 16, 24),
            (13, 15, 26, 6), (17, 29, 16, 24), (13, 15, 26, 6))
    for d in range(5):
        for r in rots[d]:
            x0 = x0 + x1
            x1 = _rotl(x1, r) ^ x0
        x0 = x0 + ks[(d + 1) % 3]
        x1 = x1 + ks[(d + 2) % 3] + jnp.int32(d + 1)
    return x0 ^ x1


def _bits_to_unit_float(bits):
    fb = lax.shift_right_logical(bits, jnp.int32(9)) | jnp.int32(_F32_ONE_BITS)
    return lax.bitcast_convert_type(fb, jnp.float32) - jnp.float32(1.0)


def _sc_bits(keys, V, vpad, row0):
    mesh = plsc.VectorSubcoreMesh(core_axis_name="c", subcore_axis_name="s")
    nstripes = vpad // 128
    per_w = (nstripes + 31) // 32

    @functools.partial(
        pl.kernel,
        out_type=jax.ShapeDtypeStruct((_SC_ROWS, vpad), jnp.int32),
        mesh=mesh,
        scratch_types=[
            pltpu.VMEM((_SC_ROWS, 128), jnp.int32),
            pltpu.VMEM((2, 16), jnp.int32),
        ],
    )
    def k(keys_hbm, out_hbm, buf, kbuf):
        wid = lax.axis_index("s") * 2 + lax.axis_index("c")
        pltpu.sync_copy(keys_hbm, kbuf)
        k0 = kbuf.at[0][...]
        k1 = kbuf.at[1][...]
        lane = lax.iota(jnp.int32, 16)

        @pl.loop(0, per_w)
        def _stripe(it):
            s = wid + it * 32

            @pl.when(s < nstripes)
            def _do():
                c0 = pl.multiple_of(s * 128, 128)

                @pl.loop(0, _SC_ROWS)
                def _row(r):
                    base = (row0 + r) * V + c0

                    for u in range(8):
                        ctr = (base + u * 16) + lane
                        buf.at[r, pl.ds(u * 16, 16)][...] = _threefry2x32(
                            k0, k1, ctr)

                pltpu.sync_copy(buf, out_hbm.at[:, pl.ds(c0, 128)])

    return k(keys)


def _make_tc_body(V, nch, blk0, with_bits):
    def _body(scal_ref, logits_ref, target_ref, *rest):
        if with_bits:
            bits_ref, out_ref = rest
        else:
            (out_ref,) = rest
        i = pl.program_id(0) + blk0
        R = _ROWS
        W = _W
        k0 = scal_ref[0]
        k1 = scal_ref[1]
        rowoff = (lax.broadcasted_iota(jnp.int32, (R, 1), 0) + i * R) * V
        lane = lax.broadcasted_iota(jnp.int32, (R, W), 1)

        def score_of(j, masked):
            x = logits_ref[:, pl.ds(j * W, W)]
            col = lane + j * W
            if with_bits:
                bits = bits_ref[:, pl.ds(j * W, W)]
            else:
                bits = _threefry2x32(k0, k1, rowoff + col)
            floats = _bits_to_unit_float(bits)
            u = jnp.maximum(_TINY, floats * (jnp.float32(1.0) - _TINY) + _TINY)
            g = -jnp.log(-jnp.log(u))
            score = x + g
            if masked:
                score = jnp.where(col < V, score, _NEG_INF)
            return score, col

        def merge(carry, score, col):
            acc_s, acc_c = carry
            take = score > acc_s
            return (jnp.where(take, score, acc_s),
                    jnp.where(take, col, acc_c))

        def chunk2(jj, carry):
            s0, c0 = score_of(jj * 2, False)
            carry = merge(carry, s0, c0)
            s1, c1 = score_of(jj * 2 + 1, False)
            return merge(carry, s1, c1)

        acc0 = (jnp.full((R, W), _NEG_INF, jnp.float32),
                jnp.full((R, W), _INT_MAX, jnp.int32))
        carry = lax.fori_loop(0, (nch - 1) // 2, chunk2, acc0)
        for j in range(2 * ((nch - 1) // 2), nch):
            s, c = score_of(j, masked=(j == nch - 1))
            carry = merge(carry, s, c)
        a_s, a_c = carry

        best = jnp.max(a_s, axis=1, keepdims=True)
        idx = jnp.min(jnp.where(a_s == best, a_c, _INT_MAX),
                      axis=1, keepdims=True)
        sample = idx.astype(jnp.float32)

        rctr = lax.broadcasted_iota(jnp.int32, (R, 1), 0) + i * R
        cbits = _threefry2x32(scal_ref[2], scal_ref[3], rctr)
        cp = jnp.maximum(jnp.float32(0.0), _bits_to_unit_float(cbits))

        t = target_ref[...]
        tcol = lax.broadcasted_iota(jnp.int32, t.shape, 1)
        tgt = jnp.sum(jnp.where(tcol == scal_ref[4], t, jnp.float32(0.0)),
                      axis=1, keepdims=True)

        thr = lax.bitcast_convert_type(scal_ref[5], jnp.float32)
        out_ref[...] = jnp.where(cp < thr, tgt, sample)

    return _body


def kernel(target, logits, step, summary_step):
    B, V = logits.shape
    T = target.shape[1]
    tc_rows = B - _SC_ROWS
    tc_blocks = tc_rows // _ROWS
    sc_blocks = _SC_ROWS // _ROWS

    skd = lax.bitcast_convert_type(
        jax.random.key_data(jax.random.fold_in(jax.random.key(42), summary_step)),
        jnp.int32)
    ckd = lax.bitcast_convert_type(
        jax.random.key_data(jax.random.fold_in(jax.random.key(7), step)),
        jnp.int32)
    stepf = jnp.asarray(step, jnp.float32)
    thr = jnp.float32(100.0) / (jnp.float32(100.0) + jnp.exp(stepf / jnp.float32(100.0)))
    scalars = jnp.concatenate([
        skd.reshape(2), ckd.reshape(2),
        jnp.asarray(step, jnp.int32).reshape(1),
        lax.bitcast_convert_type(thr, jnp.int32).reshape(1),
    ])

    nch = pl.cdiv(V, _W)
    vpad = nch * _W

    keys16 = jnp.tile(skd.reshape(2, 1), (1, 16))
    sc_bits = _sc_bits(keys16, V, vpad, tc_rows)

    out_a = pl.pallas_call(
        _make_tc_body(V, nch, 0, False),
        grid=(tc_blocks,),
        in_specs=[
            pl.BlockSpec(memory_space=pltpu.SMEM),
            pl.BlockSpec((_ROWS, vpad), lambda i: (i, 0)),
            pl.BlockSpec((_ROWS, T), lambda i: (i, 0)),
        ],
        out_specs=pl.BlockSpec((_ROWS, 1), lambda i: (i, 0)),
        out_shape=jax.ShapeDtypeStruct((tc_rows, 1), jnp.float32),
    )(scalars, logits, target)

    out_b = pl.pallas_call(
        _make_tc_body(V, nch, tc_blocks, True),
        grid=(sc_blocks,),
        in_specs=[
            pl.BlockSpec(memory_space=pltpu.SMEM),
            pl.BlockSpec((_ROWS, vpad), lambda i, b=tc_blocks: (i + b, 0)),
            pl.BlockSpec((_ROWS, T), lambda i, b=tc_blocks: (i + b, 0)),
            pl.BlockSpec((_ROWS, vpad), lambda i: (i, 0)),
        ],
        out_specs=pl.BlockSpec((_ROWS, 1), lambda i: (i, 0)),
        out_shape=jax.ShapeDtypeStruct((_SC_ROWS, 1), jnp.float32),
    )(scalars, logits, target, sc_bits)

    return jnp.concatenate([out_a, out_b], axis=0)


# speedup vs baseline: 3.5004x; 1.0154x over previous
import functools

import jax
import jax.numpy as jnp
import numpy as np
from jax import lax
from jax.experimental import pallas as pl
from jax.experimental.pallas import tpu as pltpu
from jax.experimental.pallas import tpu_sc as plsc

_ROWS = 8
_W = 1024
_SC_ROWS = 40
_SC_CW = 2000
_THREEFRY_C = 0x1BD11BDA
_F32_ONE_BITS = 0x3F800000
_TINY = np.float32(np.finfo(np.float32).tiny)
_NEG_INF = np.float32(-np.inf)
_INT_MAX = np.int32(2**31 - 1)


def _rotl(x, r):
    return (x << jnp.int32(r)) | lax.shift_right_logical(x, jnp.int32(32 - r))


def _threefry2x32(k0, k1, c1):
    k2 = k0 ^ k1 ^ jnp.int32(_THREEFRY_C)
    x0 = k0
    x1 = c1 + k1
    ks = (k0, k1, k2)
    rots = ((13, 15, 26, 6), (17, 29, 16, 24),
            (13, 15, 26, 6), (17, 29, 16, 24), (13, 15, 26, 6))
    for d in range(5):
        for r in rots[d]:
            x0 = x0 + x1
            x1 = _rotl(x1, r) ^ x0
        x0 = x0 + ks[(d + 1) % 3]
        x1 = x1 + ks[(d + 2) % 3] + jnp.int32(d + 1)
    return x0 ^ x1


def _bits_to_unit_float(bits):
    fb = lax.shift_right_logical(bits, jnp.int32(9)) | jnp.int32(_F32_ONE_BITS)
    return lax.bitcast_convert_type(fb, jnp.float32) - jnp.float32(1.0)


def _sc_bits(keys, V, vpad, row0):
    mesh = plsc.VectorSubcoreMesh(core_axis_name="c", subcore_axis_name="s")
    nstripes = vpad // 128
    per_w = (nstripes + 31) // 32

    @functools.partial(
        pl.kernel,
        out_type=jax.ShapeDtypeStruct((_SC_ROWS, vpad), jnp.int32),
        mesh=mesh,
        scratch_types=[
            pltpu.VMEM((_SC_ROWS, 128), jnp.int32),
            pltpu.VMEM((2, 16), jnp.int32),
        ],
    )
    def k(keys_hbm, out_hbm, buf, kbuf):
        wid = lax.axis_index("s") * 2 + lax.axis_index("c")
        pltpu.sync_copy(keys_hbm, kbuf)
        k0 = kbuf.at[0][...]
        k1 = kbuf.at[1][...]
        lane = lax.iota(jnp.int32, 16)

        @pl.loop(0, per_w)
        def _stripe(it):
            s = wid + it * 32

            @pl.when(s < nstripes)
            def _do():
                c0 = pl.multiple_of(s * 128, 128)

                @pl.loop(0, _SC_ROWS)
                def _row(r):
                    base = (row0 + r) * V + c0

                    for u in range(8):
                        ctr = (base + u * 16) + lane
                        buf.at[r, pl.ds(u * 16, 16)][...] = _threefry2x32(
                            k0, k1, ctr)

                pltpu.sync_copy(buf, out_hbm.at[:, pl.ds(c0, 128)])

    return k(keys)


def _make_tc_body(V, W, nch, blk0, with_bits):
    def _body(scal_ref, logits_ref, target_ref, *rest):
        if with_bits:
            bits_ref, out_ref = rest
        else:
            (out_ref,) = rest
        i = pl.program_id(0) + blk0
        R = _ROWS
        k0 = scal_ref[0]
        k1 = scal_ref[1]
        rowoff = (lax.broadcasted_iota(jnp.int32, (R, 1), 0) + i * R) * V
        lane = lax.broadcasted_iota(jnp.int32, (R, W), 1)

        def score_of(j, masked):
            x = logits_ref[:, pl.ds(j * W, W)]
            if with_bits:
                bits = bits_ref[:, pl.ds(j * W, W)]
            else:
                bits = _threefry2x32(k0, k1, (rowoff + j * W) + lane)
            floats = _bits_to_unit_float(bits)
            u = floats + _TINY
            g = -jnp.log(-jnp.log(u))
            score = x + g
            if masked:
                score = jnp.where(lane + j * W < V, score, _NEG_INF)
            return score

        def merge(carry, score, j):
            acc_s, acc_j = carry
            take = score > acc_s
            return (jnp.where(take, score, acc_s),
                    jnp.where(take, j, acc_j))

        def chunk2(jj, carry):
            carry = merge(carry, score_of(jj * 2, False), jj * 2)
            return merge(carry, score_of(jj * 2 + 1, False), jj * 2 + 1)

        acc0 = (jnp.full((R, W), _NEG_INF, jnp.float32),
                jnp.zeros((R, W), jnp.int32))
        carry = lax.fori_loop(0, (nch - 1) // 2, chunk2, acc0)
        for j in range(2 * ((nch - 1) // 2), nch):
            carry = merge(carry, score_of(j, masked=(j == nch - 1)), j)
        a_s, a_j = carry
        a_c = a_j * W + lane

        best = jnp.max(a_s, axis=1, keepdims=True)
        idx = jnp.min(jnp.where(a_s == best, a_c, _INT_MAX),
                      axis=1, keepdims=True)
        sample = idx.astype(jnp.float32)

        rctr = lax.broadcasted_iota(jnp.int32, (R, 1), 0) + i * R
        cbits = _threefry2x32(scal_ref[2], scal_ref[3], rctr)
        cp = jnp.maximum(jnp.float32(0.0), _bits_to_unit_float(cbits))

        t = target_ref[...]
        tcol = lax.broadcasted_iota(jnp.int32, t.shape, 1)
        tgt = jnp.sum(jnp.where(tcol == scal_ref[4], t, jnp.float32(0.0)),
                      axis=1, keepdims=True)

        thr = lax.bitcast_convert_type(scal_ref[5], jnp.float32)
        out_ref[...] = jnp.where(cp < thr, tgt, sample)

    return _body


def kernel(target, logits, step, summary_step):
    B, V = logits.shape
    T = target.shape[1]
    tc_rows = B - _SC_ROWS
    tc_blocks = tc_rows // _ROWS
    sc_blocks = _SC_ROWS // _ROWS

    skd = lax.bitcast_convert_type(
        jax.random.key_data(jax.random.fold_in(jax.random.key(42), summary_step)),
        jnp.int32)
    ckd = lax.bitcast_convert_type(
        jax.random.key_data(jax.random.fold_in(jax.random.key(7), step)),
        jnp.int32)
    stepf = jnp.asarray(step, jnp.float32)
    thr = jnp.float32(100.0) / (jnp.float32(100.0) + jnp.exp(stepf / jnp.float32(100.0)))
    scalars = jnp.concatenate([
        skd.reshape(2), ckd.reshape(2),
        jnp.asarray(step, jnp.int32).reshape(1),
        lax.bitcast_convert_type(thr, jnp.int32).reshape(1),
    ])

    nch = pl.cdiv(V, _W)
    vpad = nch * _W

    keys16 = jnp.tile(skd.reshape(2, 1), (1, 16))
    sc_bits = _sc_bits(keys16, V, vpad, tc_rows)

    wb = 2048
    nch_b = vpad // wb

    out_a = pl.pallas_call(
        _make_tc_body(V, _W, nch, 0, False),
        grid=(tc_blocks,),
        in_specs=[
            pl.BlockSpec(memory_space=pltpu.SMEM),
            pl.BlockSpec((_ROWS, vpad), lambda i: (i, 0)),
            pl.BlockSpec((_ROWS, T), lambda i: (i, 0)),
        ],
        out_specs=pl.BlockSpec((_ROWS, 1), lambda i: (i, 0)),
        out_shape=jax.ShapeDtypeStruct((tc_rows, 1), jnp.float32),
    )(scalars, logits, target)

    out_b = pl.pallas_call(
        _make_tc_body(V, wb, nch_b, tc_blocks, True),
        grid=(sc_blocks,),
        in_specs=[
            pl.BlockSpec(memory_space=pltpu.SMEM),
            pl.BlockSpec((_ROWS, vpad), lambda i, b=tc_blocks: (i + b, 0)),
            pl.BlockSpec((_ROWS, T), lambda i, b=tc_blocks: (i + b, 0)),
            pl.BlockSpec((_ROWS, vpad), lambda i: (i, 0)),
        ],
        out_specs=pl.BlockSpec((_ROWS, 1), lambda i: (i, 0)),
        out_shape=jax.ShapeDtypeStruct((_SC_ROWS, 1), jnp.float32),
    )(scalars, logits, target, sc_bits)

    return jnp.concatenate([out_a, out_b], axis=0)
